# Initial kernel scaffold; baseline (speedup 1.0000x reference)
import jax
import jax.numpy as jnp
from jax.experimental import pallas as pl


def kernel(src, W_word, W_f1, W_f2, W_mlp, b_mlp, pe):
    raise NotImplementedError("write your pallas kernel here")


# speedup vs baseline: 2.9876x; 2.9876x over previous
import functools
import math

import jax
import jax.numpy as jnp
from jax import lax
from jax.experimental import pallas as pl
from jax.experimental.pallas import tpu as pltpu
from jax.experimental.pallas import tpu_sc as plsc


def _make_sc_gather(V, D, Vf, Df, N):
    info = plsc.get_sparse_core_info()
    NC, NS = info.num_cores, info.num_subcores
    NW = NC * NS
    assert N % NW == 0
    T = N // NW
    CH = 32
    NCHUNK = T // CH
    FCH = 128
    NFCH = T // FCH

    mesh = plsc.VectorSubcoreMesh(core_axis_name="c", subcore_axis_name="s")

    @functools.partial(
        pl.kernel,
        mesh=mesh,
        out_type=[
            jax.ShapeDtypeStruct((N, D), jnp.float32),
            jax.ShapeDtypeStruct((N, Df), jnp.float32),
            jax.ShapeDtypeStruct((N, Df), jnp.float32),
        ],
        scratch_types=[
            pltpu.VMEM((T,), jnp.int32),
            pltpu.VMEM((T,), jnp.int32),
            pltpu.VMEM((T,), jnp.int32),
            pltpu.VMEM((CH, D), jnp.float32),
            pltpu.VMEM((CH, D), jnp.float32),
            pltpu.VMEM((FCH, Df), jnp.float32),
            pltpu.SemaphoreType.DMA,
            pltpu.SemaphoreType.DMA,
            pltpu.SemaphoreType.DMA,
        ],
    )
    def sc_gather(w_hbm, f1_hbm, f2_hbm, i0_hbm, i1_hbm, i2_hbm,
                  e0_hbm, e1_hbm, e2_hbm,
                  i0_v, i1_v, i2_v, wbuf0, wbuf1, fbuf, sem0, sem1, semf):
        wid = lax.axis_index("s") * NC + lax.axis_index("c")
        base = wid * T
        pltpu.sync_copy(i0_hbm.at[pl.ds(base, T)], i0_v)
        pltpu.sync_copy(i1_hbm.at[pl.ds(base, T)], i1_v)
        pltpu.sync_copy(i2_hbm.at[pl.ds(base, T)], i2_v)

        for idx_v, tbl, out in ((i1_v, f1_hbm, e1_hbm), (i2_v, f2_hbm, e2_hbm)):
            for j in range(NFCH):
                pltpu.async_copy(
                    tbl.at[idx_v.at[pl.ds(j * FCH, FCH)]], fbuf, semf).wait()
                pltpu.sync_copy(fbuf, out.at[pl.ds(base + j * FCH, FCH)])

        bufs = (wbuf0, wbuf1)
        sems = (sem0, sem1)

        def start(cidx):
            b = cidx % 2
            return pltpu.async_copy(
                w_hbm.at[i0_v.at[pl.ds(cidx * CH, CH)]], bufs[b], sems[b])

        cps = [None, None]
        cps[0] = start(0)
        for cidx in range(NCHUNK):
            nxt = cidx + 1
            if nxt < NCHUNK:
                cps[nxt % 2] = start(nxt)
            cps[cidx % 2].wait()
            pltpu.sync_copy(bufs[cidx % 2],
                            e0_hbm.at[pl.ds(base + cidx * CH, CH)])

    return sc_gather


def _tc_body(e0_ref, e1_ref, e2_ref, w0_ref, w1_ref, w2_ref, b_ref, pe_ref,
             o_ref, *, scale):
    acc = jnp.dot(e0_ref[...], w0_ref[...], preferred_element_type=jnp.float32)
    acc += jnp.dot(e1_ref[...], w1_ref[...], preferred_element_type=jnp.float32)
    acc += jnp.dot(e2_ref[...], w2_ref[...], preferred_element_type=jnp.float32)
    h = jnp.maximum(acc + b_ref[...], 0.0)
    o_ref[...] = h * scale + pe_ref[...]


def _tc_mlp(e0, e1, e2, W0, W1, W2, b, peL, L):
    N, D = e0.shape
    Df = e1.shape[1]
    bm = 512
    grid = (N // bm,)
    nblk_pe = L // bm
    scale = math.sqrt(D)
    return pl.pallas_call(
        functools.partial(_tc_body, scale=scale),
        grid=grid,
        in_specs=[
            pl.BlockSpec((bm, D), lambda i: (i, 0)),
            pl.BlockSpec((bm, Df), lambda i: (i, 0)),
            pl.BlockSpec((bm, Df), lambda i: (i, 0)),
            pl.BlockSpec((D, D), lambda i: (0, 0)),
            pl.BlockSpec((Df, D), lambda i: (0, 0)),
            pl.BlockSpec((Df, D), lambda i: (0, 0)),
            pl.BlockSpec((1, D), lambda i: (0, 0)),
            pl.BlockSpec((bm, D), lambda i: (i % nblk_pe, 0)),
        ],
        out_specs=pl.BlockSpec((bm, D), lambda i: (i, 0)),
        out_shape=jax.ShapeDtypeStruct((N, D), jnp.float32),
        compiler_params=pltpu.CompilerParams(
            dimension_semantics=("arbitrary",),
        ),
    )(e0, e1, e2, W0, W1, W2, b, peL)


def kernel(src, W_word, W_f1, W_f2, W_mlp, b_mlp, pe):
    B, L, _ = src.shape
    N = B * L
    V, D = W_word.shape
    Vf, Df = W_f1.shape

    idx = src.reshape(N, 3)
    i0 = idx[:, 0]
    i1 = idx[:, 1]
    i2 = idx[:, 2]

    Dfp = 128
    W_f1p = jnp.pad(W_f1, ((0, 0), (0, Dfp - Df)))
    W_f2p = jnp.pad(W_f2, ((0, 0), (0, Dfp - Df)))

    sc_gather = _make_sc_gather(V, D, Vf, Dfp, N)
    e0, e1, e2 = sc_gather(W_word, W_f1p, W_f2p, i0, i1, i2)

    W0 = W_mlp[:D]
    W1 = jnp.pad(W_mlp[D:D + Df], ((0, Dfp - Df), (0, 0)))
    W2 = jnp.pad(W_mlp[D + Df:], ((0, Dfp - Df), (0, 0)))
    b = b_mlp.reshape(1, D)

    out = _tc_mlp(e0, e1, e2, W0, W1, W2, b, pe[:L], L)
    return out.reshape(B, L, D)


# speedup vs baseline: 3.0260x; 1.0128x over previous
import functools
import math

import jax
import jax.numpy as jnp
from jax import lax
from jax.experimental import pallas as pl
from jax.experimental.pallas import tpu as pltpu
from jax.experimental.pallas import tpu_sc as plsc


def _make_sc_gather(V, D, Vf, Df, N):
    info = plsc.get_sparse_core_info()
    NC, NS = info.num_cores, info.num_subcores
    NW = NC * NS
    assert N % NW == 0
    T = N // NW
    CH = 32
    NCHUNK = T // CH
    FCH = 128
    NFCH = T // FCH

    mesh = plsc.VectorSubcoreMesh(core_axis_name="c", subcore_axis_name="s")

    @functools.partial(
        pl.kernel,
        mesh=mesh,
        out_type=[
            jax.ShapeDtypeStruct((N, D), jnp.float32),
            jax.ShapeDtypeStruct((N, Df), jnp.float32),
            jax.ShapeDtypeStruct((N, Df), jnp.float32),
        ],
        scratch_types=[
            pltpu.VMEM((T,), jnp.int32),
            pltpu.VMEM((T,), jnp.int32),
            pltpu.VMEM((T,), jnp.int32),
            pltpu.VMEM((CH, D), jnp.float32),
            pltpu.VMEM((CH, D), jnp.float32),
            pltpu.VMEM((FCH, Df), jnp.float32),
            pltpu.SemaphoreType.DMA,
            pltpu.SemaphoreType.DMA,
            pltpu.SemaphoreType.DMA,
        ],
    )
    def sc_gather(w_hbm, f1_hbm, f2_hbm, i0_hbm, i1_hbm, i2_hbm,
                  e0_hbm, e1_hbm, e2_hbm,
                  i0_v, i1_v, i2_v, wbuf0, wbuf1, fbuf, sem0, sem1, semf):
        wid = lax.axis_index("s") * NC + lax.axis_index("c")
        base = wid * T
        pltpu.sync_copy(i0_hbm.at[pl.ds(base, T)], i0_v)
        pltpu.sync_copy(i1_hbm.at[pl.ds(base, T)], i1_v)
        pltpu.sync_copy(i2_hbm.at[pl.ds(base, T)], i2_v)

        for idx_v, tbl, out in ((i1_v, f1_hbm, e1_hbm), (i2_v, f2_hbm, e2_hbm)):
            for j in range(NFCH):
                pltpu.async_copy(
                    tbl.at[idx_v.at[pl.ds(j * FCH, FCH)]], fbuf, semf).wait()
                pltpu.sync_copy(fbuf, out.at[pl.ds(base + j * FCH, FCH)])

        bufs = (wbuf0, wbuf1)
        sems = (sem0, sem1)

        def start(cidx):
            b = cidx % 2
            return pltpu.async_copy(
                w_hbm.at[i0_v.at[pl.ds(cidx * CH, CH)]], bufs[b], sems[b])

        cps = [None, None]
        cps[0] = start(0)
        for cidx in range(NCHUNK):
            nxt = cidx + 1
            if nxt < NCHUNK:
                cps[nxt % 2] = start(nxt)
            cps[cidx % 2].wait()
            pltpu.sync_copy(bufs[cidx % 2],
                            e0_hbm.at[pl.ds(base + cidx * CH, CH)])

    return sc_gather


def _tc_body(e0_ref, e1_ref, e2_ref, w0_ref, w1_ref, w2_ref, b_ref, pe_ref,
             o_ref, *, scale):
    e0 = e0_ref[...].astype(jnp.bfloat16)
    e1 = e1_ref[...].astype(jnp.bfloat16)
    e2 = e2_ref[...].astype(jnp.bfloat16)
    acc = jnp.dot(e0, w0_ref[...], preferred_element_type=jnp.float32)
    acc += jnp.dot(e1, w1_ref[...], preferred_element_type=jnp.float32)
    acc += jnp.dot(e2, w2_ref[...], preferred_element_type=jnp.float32)
    h = jnp.maximum(acc + b_ref[...], 0.0)
    o_ref[...] = h * scale + pe_ref[...]


def _tc_mlp(e0, e1, e2, W0, W1, W2, b, peL, L):
    N, D = e0.shape
    Df = e1.shape[1]
    bm = 512
    grid = (N // bm,)
    nblk_pe = L // bm
    scale = math.sqrt(D)
    return pl.pallas_call(
        functools.partial(_tc_body, scale=scale),
        grid=grid,
        in_specs=[
            pl.BlockSpec((bm, D), lambda i: (i, 0)),
            pl.BlockSpec((bm, Df), lambda i: (i, 0)),
            pl.BlockSpec((bm, Df), lambda i: (i, 0)),
            pl.BlockSpec((D, D), lambda i: (0, 0)),
            pl.BlockSpec((Df, D), lambda i: (0, 0)),
            pl.BlockSpec((Df, D), lambda i: (0, 0)),
            pl.BlockSpec((1, D), lambda i: (0, 0)),
            pl.BlockSpec((bm, D), lambda i: (i % nblk_pe, 0)),
        ],
        out_specs=pl.BlockSpec((bm, D), lambda i: (i, 0)),
        out_shape=jax.ShapeDtypeStruct((N, D), jnp.float32),
        compiler_params=pltpu.CompilerParams(
            dimension_semantics=("arbitrary",),
        ),
    )(e0, e1, e2, W0, W1, W2, b, peL)


def kernel(src, W_word, W_f1, W_f2, W_mlp, b_mlp, pe):
    B, L, _ = src.shape
    N = B * L
    V, D = W_word.shape
    Vf, Df = W_f1.shape

    idx = src.reshape(N, 3)
    i0 = idx[:, 0]
    i1 = idx[:, 1]
    i2 = idx[:, 2]

    Dfp = 128
    W_f1p = jnp.pad(W_f1, ((0, 0), (0, Dfp - Df)))
    W_f2p = jnp.pad(W_f2, ((0, 0), (0, Dfp - Df)))

    sc_gather = _make_sc_gather(V, D, Vf, Dfp, N)
    e0, e1, e2 = sc_gather(W_word, W_f1p, W_f2p, i0, i1, i2)

    W0 = W_mlp[:D].astype(jnp.bfloat16)
    W1 = jnp.pad(W_mlp[D:D + Df], ((0, Dfp - Df), (0, 0))).astype(jnp.bfloat16)
    W2 = jnp.pad(W_mlp[D + Df:], ((0, Dfp - Df), (0, 0))).astype(jnp.bfloat16)
    b = b_mlp.reshape(1, D)

    out = _tc_mlp(e0, e1, e2, W0, W1, W2, b, pe[:L], L)
    return out.reshape(B, L, D)
